# baseline (speedup 1.0000x reference)
import functools

import jax
import jax.numpy as jnp
from jax import lax
from jax.experimental import pallas as pl
from jax.experimental.pallas import tpu as pltpu
from jax.experimental.pallas import tpu_sc as plsc

N = 3200000
S = 1024
EPS = 1e-5
L = 16
NC, NS = 2, 16
NW = NC * NS
CHB = 3200
NCHT = N // CHB
VPC = CHB // L
NFULL = NCHT // NW
NEXTRA = NCHT % NW
SB = 256
SSTR = S + 1

_mesh = plsc.VectorSubcoreMesh(core_axis_name="c", subcore_axis_name="s")
_params = pltpu.CompilerParams(needs_layout_passes=False)


def _fast_norm(n2):
    i = lax.bitcast_convert_type(n2, jnp.int32)
    i = jnp.full((L,), 0x5F3759DF, jnp.int32) - lax.shift_right_logical(i, 1)
    y = lax.bitcast_convert_type(i, jnp.float32)
    ah = n2 * jnp.full((L,), 0.5, jnp.float32)
    c15 = jnp.full((L,), 1.5, jnp.float32)
    y = y * (c15 - ah * y * y)
    return n2 * y


@functools.partial(
    pl.kernel,
    mesh=_mesh,
    out_type=[
        jax.ShapeDtypeStruct((NW, S), jnp.float32),
        jax.ShapeDtypeStruct((NW, S), jnp.float32),
    ],
    scratch_types=[
        pltpu.VMEM((3, CHB), jnp.float32),
        pltpu.VMEM((3, CHB), jnp.float32),
        pltpu.VMEM((CHB,), jnp.int32),
        pltpu.VMEM((CHB,), jnp.int32),
        pltpu.VMEM((L * SSTR,), jnp.float32),
        pltpu.VMEM((L * SSTR,), jnp.float32),
        pltpu.VMEM((S,), jnp.float32),
        pltpu.VMEM((S,), jnp.float32),
        pltpu.SemaphoreType.DMA,
        pltpu.SemaphoreType.DMA,
        pltpu.SemaphoreType.DMA,
        pltpu.SemaphoreType.DMA,
    ],
    compiler_params=_params,
)
def _pass1(pos_hbm, batch_hbm, psum_hbm, pcnt_hbm,
           bufa, bufb, bba, bbb, accs, accc, reds, redc,
           spa, sba, spb, sbb):
    wid = lax.axis_index("s") * NC + lax.axis_index("c")
    lanes = lax.iota(jnp.int32, L)
    laneoff = lanes * SSTR
    zero = jnp.zeros((L,), jnp.float32)
    ones = jnp.ones((L,), jnp.float32)
    n_my = jnp.int32(NFULL) + jnp.where(wid < NEXTRA, 1, 0).astype(jnp.int32)

    @plsc.parallel_loop(0, SSTR, unroll=5)
    def _(i):
        accs[pl.ds(i * L, L)] = zero
        accc[pl.ds(i * L, L)] = zero

    def start(k, bufp, bufb2, semp, semb):
        base = (wid + k * NW) * CHB
        pltpu.async_copy(pos_hbm.at[:, pl.ds(base, CHB)], bufp, semp)
        pltpu.async_copy(batch_hbm.at[pl.ds(base, CHB)], bufb2, semb)

    def wait(bufp, bufb2, semp, semb):
        pltpu.make_async_copy(pos_hbm.at[:, pl.ds(0, CHB)], bufp, semp).wait()
        pltpu.make_async_copy(batch_hbm.at[pl.ds(0, CHB)], bufb2, semb).wait()

    def compute(bufp, bufb2):
        @plsc.parallel_loop(0, VPC, unroll=16)
        def _(v):
            o = v * L
            b = bufb2[pl.ds(o, L)]
            x = bufp[0, pl.ds(o, L)]
            y = bufp[1, pl.ds(o, L)]
            z = bufp[2, pl.ds(o, L)]
            nrm = _fast_norm(x * x + y * y + z * z)
            idx = b + laneoff
            plsc.addupdate_scatter(accs, [idx], nrm)
            plsc.addupdate_scatter(accc, [idx], ones)

    start(0, bufa, bba, spa, sba)

    def pair_body(j, carry):
        k1 = 2 * j + 1
        k2 = 2 * j + 2

        @pl.when(k1 < n_my)
        def _():
            start(k1, bufb, bbb, spb, sbb)

        wait(bufa, bba, spa, sba)
        compute(bufa, bba)

        @pl.when(k2 < n_my)
        def _():
            start(k2, bufa, bba, spa, sba)

        @pl.when(k1 < n_my)
        def _():
            wait(bufb, bbb, spb, sbb)
            compute(bufb, bbb)

        return carry

    lax.fori_loop(0, (NFULL + 1) // 2, pair_body, 0)

    def red_body(g, carry):
        sbase = g * L
        ssum = accs[pl.ds(sbase, L)]
        scnt = accc[pl.ds(sbase, L)]
        for c in range(1, L):
            ssum = ssum + accs[pl.ds(c * SSTR + sbase, L)]
            scnt = scnt + accc[pl.ds(c * SSTR + sbase, L)]
        reds[pl.ds(sbase, L)] = ssum
        redc[pl.ds(sbase, L)] = scnt
        return carry

    lax.fori_loop(0, S // L, red_body, 0)
    pltpu.sync_copy(reds, psum_hbm.at[wid])
    pltpu.sync_copy(redc, pcnt_hbm.at[wid])


@functools.partial(
    pl.kernel,
    mesh=_mesh,
    out_type=jax.ShapeDtypeStruct((4, N), jnp.float32),
    scratch_types=[
        pltpu.VMEM((NW, S), jnp.float32),
        pltpu.VMEM((NW, S), jnp.float32),
        pltpu.VMEM((S,), jnp.float32),
        pltpu.VMEM((L,), jnp.float32),
        pltpu.VMEM((3, CHB), jnp.float32),
        pltpu.VMEM((3, CHB), jnp.float32),
        pltpu.VMEM((CHB,), jnp.int32),
        pltpu.VMEM((CHB,), jnp.int32),
        pltpu.VMEM((4, CHB), jnp.float32),
        pltpu.VMEM((4, CHB), jnp.float32),
        pltpu.SemaphoreType.DMA,
        pltpu.SemaphoreType.DMA,
        pltpu.SemaphoreType.DMA,
        pltpu.SemaphoreType.DMA,
        pltpu.SemaphoreType.DMA,
        pltpu.SemaphoreType.DMA,
        pltpu.SemaphoreType.DMA,
        pltpu.SemaphoreType.DMA,
    ],
    compiler_params=_params,
)
def _pass2(pos_hbm, batch_hbm, w_hbm, psum_hbm, pcnt_hbm, out_hbm,
           psb, pcb, rbuf, wbuf, bufa, bufb, bba, bbb, oba, obb,
           spa, sba, spb, sbb, soa, sob, sps, spc):
    wid = lax.axis_index("s") * NC + lax.axis_index("c")
    onev = jnp.ones((L,), jnp.float32)
    epsv = jnp.full((L,), EPS, jnp.float32)
    n_my = jnp.int32(NFULL) + jnp.where(wid < NEXTRA, 1, 0).astype(jnp.int32)

    pltpu.async_copy(psum_hbm, psb, sps)
    pltpu.async_copy(pcnt_hbm, pcb, spc)
    pltpu.sync_copy(w_hbm, wbuf)
    w = wbuf[pl.ds(0, L)]

    def start(k, bufp, bufb2, semp, semb):
        base = (wid + k * NW) * CHB
        pltpu.async_copy(pos_hbm.at[:, pl.ds(base, CHB)], bufp, semp)
        pltpu.async_copy(batch_hbm.at[pl.ds(base, CHB)], bufb2, semb)

    def wait_in(bufp, bufb2, semp, semb):
        pltpu.make_async_copy(pos_hbm.at[:, pl.ds(0, CHB)], bufp, semp).wait()
        pltpu.make_async_copy(batch_hbm.at[pl.ds(0, CHB)], bufb2, semb).wait()

    def start_out(k, obuf, semo):
        base = (wid + k * NW) * CHB
        pltpu.async_copy(obuf, out_hbm.at[:, pl.ds(base, CHB)], semo)

    def wait_out(obuf, semo):
        pltpu.make_async_copy(obuf, out_hbm.at[:, pl.ds(0, CHB)], semo).wait()

    def compute(bufp, bufb2, obuf):
        @plsc.parallel_loop(0, VPC, unroll=16)
        def _(v):
            o = v * L
            b = bufb2[pl.ds(o, L)]
            r = plsc.load_gather(rbuf, [b])
            obuf[0, pl.ds(o, L)] = bufp[0, pl.ds(o, L)] * r
            obuf[1, pl.ds(o, L)] = bufp[1, pl.ds(o, L)] * r
            obuf[2, pl.ds(o, L)] = bufp[2, pl.ds(o, L)] * r

    start(0, bufa, bba, spa, sba)

    pltpu.make_async_copy(psum_hbm, psb, sps).wait()
    pltpu.make_async_copy(pcnt_hbm, pcb, spc).wait()

    def r_body(g, carry):
        sbase = g * L
        ssum = psb[0, pl.ds(sbase, L)]
        scnt = pcb[0, pl.ds(sbase, L)]
        for t in range(1, NW):
            ssum = ssum + psb[t, pl.ds(sbase, L)]
            scnt = scnt + pcb[t, pl.ds(sbase, L)]
        mean = ssum / jnp.maximum(scnt, onev)
        rbuf[pl.ds(sbase, L)] = w / (mean + epsv)
        return carry

    lax.fori_loop(0, S // L, r_body, 0)

    def pair_body(j, carry):
        k1 = 2 * j + 1
        k2 = 2 * j + 2

        @pl.when(k1 < n_my)
        def _():
            start(k1, bufb, bbb, spb, sbb)

        wait_in(bufa, bba, spa, sba)

        @pl.when(j > 0)
        def _():
            wait_out(oba, soa)

        compute(bufa, bba, oba)
        start_out(2 * j, oba, soa)

        @pl.when(k2 < n_my)
        def _():
            start(k2, bufa, bba, spa, sba)

        @pl.when(k1 < n_my)
        def _():
            wait_in(bufb, bbb, spb, sbb)

            @pl.when(j > 0)
            def _():
                wait_out(obb, sob)

            compute(bufb, bbb, obb)
            start_out(k1, obb, sob)

        return carry

    lax.fori_loop(0, (NFULL + 1) // 2, pair_body, 0)
    wait_out(oba, soa)
    wait_out(obb, sob)


def kernel(pos, batch, weight):
    pos_t = jnp.swapaxes(pos, 0, 1)
    wvec = jnp.broadcast_to(weight.reshape(1), (L,)).astype(jnp.float32)
    psum, pcnt = _pass1(pos_t, batch)
    out4 = _pass2(pos_t, batch, wvec, psum, pcnt)
    return jnp.swapaxes(lax.slice(out4, (0, 0), (3, N)), 0, 1)


# speedup vs baseline: 1.3228x; 1.3228x over previous
import functools

import jax
import jax.numpy as jnp
from jax import lax
from jax.experimental import pallas as pl
from jax.experimental.pallas import tpu as pltpu
from jax.experimental.pallas import tpu_sc as plsc

N = 3200000
S = 1024
EPS = 1e-5
L = 16
NC, NS = 2, 16
NW = NC * NS
CHB = 3200
NCHT = N // CHB
VPC = CHB // L
NFULL = NCHT // NW
NEXTRA = NCHT % NW
CHB1 = 6400
NCHT1 = N // CHB1
VPC1 = CHB1 // L
NFULL1 = NCHT1 // NW
NEXTRA1 = NCHT1 % NW
SB = 256
SSTR = S + 1

_mesh = plsc.VectorSubcoreMesh(core_axis_name="c", subcore_axis_name="s")
_params = pltpu.CompilerParams(needs_layout_passes=False)


def _fast_norm(n2):
    i = lax.bitcast_convert_type(n2, jnp.int32)
    i = jnp.full((L,), 0x5F3759DF, jnp.int32) - lax.shift_right_logical(i, 1)
    y = lax.bitcast_convert_type(i, jnp.float32)
    ah = n2 * jnp.full((L,), 0.5, jnp.float32)
    c15 = jnp.full((L,), 1.5, jnp.float32)
    y = y * (c15 - ah * y * y)
    return n2 * y


@functools.partial(
    pl.kernel,
    mesh=_mesh,
    out_type=[
        jax.ShapeDtypeStruct((NW, S), jnp.float32),
        jax.ShapeDtypeStruct((NW, S), jnp.float32),
    ],
    scratch_types=[
        pltpu.VMEM((3, CHB1), jnp.float32),
        pltpu.VMEM((3, CHB1), jnp.float32),
        pltpu.VMEM((CHB1,), jnp.int32),
        pltpu.VMEM((CHB1,), jnp.int32),
        pltpu.VMEM((L * SSTR,), jnp.float32),
        pltpu.VMEM((L * SSTR,), jnp.float32),
        pltpu.VMEM((S,), jnp.float32),
        pltpu.VMEM((S,), jnp.float32),
        pltpu.SemaphoreType.DMA,
        pltpu.SemaphoreType.DMA,
        pltpu.SemaphoreType.DMA,
        pltpu.SemaphoreType.DMA,
    ],
    compiler_params=_params,
)
def _pass1(pos_hbm, batch_hbm, psum_hbm, pcnt_hbm,
           bufa, bufb, bba, bbb, accs, accc, reds, redc,
           spa, sba, spb, sbb):
    wid = lax.axis_index("s") * NC + lax.axis_index("c")
    lanes = lax.iota(jnp.int32, L)
    laneoff = lanes * SSTR
    zero = jnp.zeros((L,), jnp.float32)
    ones = jnp.ones((L,), jnp.float32)
    n_my = jnp.int32(NFULL1) + jnp.where(wid < NEXTRA1, 1, 0).astype(jnp.int32)

    @plsc.parallel_loop(0, SSTR, unroll=5)
    def _(i):
        accs[pl.ds(i * L, L)] = zero
        accc[pl.ds(i * L, L)] = zero

    def start(k, bufp, bufb2, semp, semb):
        base = (wid + k * NW) * CHB1
        pltpu.async_copy(pos_hbm.at[:, pl.ds(base, CHB1)], bufp, semp)
        pltpu.async_copy(batch_hbm.at[pl.ds(base, CHB1)], bufb2, semb)

    def wait(bufp, bufb2, semp, semb):
        pltpu.make_async_copy(pos_hbm.at[:, pl.ds(0, CHB1)], bufp, semp).wait()
        pltpu.make_async_copy(batch_hbm.at[pl.ds(0, CHB1)], bufb2, semb).wait()

    def compute(bufp, bufb2):
        @plsc.parallel_loop(0, VPC1, unroll=8)
        def _(v):
            o = v * L
            b = bufb2[pl.ds(o, L)]
            x = bufp[0, pl.ds(o, L)]
            y = bufp[1, pl.ds(o, L)]
            z = bufp[2, pl.ds(o, L)]
            nrm = _fast_norm(x * x + y * y + z * z)
            idx = b + laneoff
            plsc.addupdate_scatter(accs, [idx], nrm)
            plsc.addupdate_scatter(accc, [idx], ones)

    start(0, bufa, bba, spa, sba)

    def pair_body(j, carry):
        k1 = 2 * j + 1
        k2 = 2 * j + 2

        @pl.when(k1 < n_my)
        def _():
            start(k1, bufb, bbb, spb, sbb)

        wait(bufa, bba, spa, sba)
        compute(bufa, bba)

        @pl.when(k2 < n_my)
        def _():
            start(k2, bufa, bba, spa, sba)

        @pl.when(k1 < n_my)
        def _():
            wait(bufb, bbb, spb, sbb)
            compute(bufb, bbb)

        return carry

    lax.fori_loop(0, (NFULL1 + 1) // 2, pair_body, 0)

    def red_body(g, carry):
        sbase = g * L
        ssum = accs[pl.ds(sbase, L)]
        scnt = accc[pl.ds(sbase, L)]
        for c in range(1, L):
            ssum = ssum + accs[pl.ds(c * SSTR + sbase, L)]
            scnt = scnt + accc[pl.ds(c * SSTR + sbase, L)]
        reds[pl.ds(sbase, L)] = ssum
        redc[pl.ds(sbase, L)] = scnt
        return carry

    lax.fori_loop(0, S // L, red_body, 0)
    pltpu.sync_copy(reds, psum_hbm.at[wid])
    pltpu.sync_copy(redc, pcnt_hbm.at[wid])


@functools.partial(
    pl.kernel,
    mesh=_mesh,
    out_type=jax.ShapeDtypeStruct((3, N), jnp.float32),
    scratch_types=[
        pltpu.VMEM((NW, S), jnp.float32),
        pltpu.VMEM((NW, S), jnp.float32),
        pltpu.VMEM((S,), jnp.float32),
        pltpu.VMEM((L,), jnp.float32),
        pltpu.VMEM((3, CHB), jnp.float32),
        pltpu.VMEM((3, CHB), jnp.float32),
        pltpu.VMEM((CHB,), jnp.int32),
        pltpu.VMEM((CHB,), jnp.int32),
        pltpu.VMEM((3, CHB), jnp.float32),
        pltpu.VMEM((3, CHB), jnp.float32),
        pltpu.SemaphoreType.DMA,
        pltpu.SemaphoreType.DMA,
        pltpu.SemaphoreType.DMA,
        pltpu.SemaphoreType.DMA,
        pltpu.SemaphoreType.DMA,
        pltpu.SemaphoreType.DMA,
        pltpu.SemaphoreType.DMA,
        pltpu.SemaphoreType.DMA,
    ],
    compiler_params=_params,
)
def _pass2(pos_hbm, batch_hbm, w_hbm, psum_hbm, pcnt_hbm, out_hbm,
           psb, pcb, rbuf, wbuf, bufa, bufb, bba, bbb, oba, obb,
           spa, sba, spb, sbb, soa, sob, sps, spc):
    wid = lax.axis_index("s") * NC + lax.axis_index("c")
    onev = jnp.ones((L,), jnp.float32)
    epsv = jnp.full((L,), EPS, jnp.float32)
    n_my = jnp.int32(NFULL) + jnp.where(wid < NEXTRA, 1, 0).astype(jnp.int32)

    pltpu.async_copy(psum_hbm, psb, sps)
    pltpu.async_copy(pcnt_hbm, pcb, spc)
    pltpu.sync_copy(w_hbm, wbuf)
    w = wbuf[pl.ds(0, L)]

    def start(k, bufp, bufb2, semp, semb):
        base = (wid + k * NW) * CHB
        pltpu.async_copy(pos_hbm.at[:, pl.ds(base, CHB)], bufp, semp)
        pltpu.async_copy(batch_hbm.at[pl.ds(base, CHB)], bufb2, semb)

    def wait_in(bufp, bufb2, semp, semb):
        pltpu.make_async_copy(pos_hbm.at[:, pl.ds(0, CHB)], bufp, semp).wait()
        pltpu.make_async_copy(batch_hbm.at[pl.ds(0, CHB)], bufb2, semb).wait()

    def start_out(k, obuf, semo):
        base = (wid + k * NW) * CHB
        pltpu.async_copy(obuf, out_hbm.at[:, pl.ds(base, CHB)], semo)

    def wait_out(obuf, semo):
        pltpu.make_async_copy(obuf, out_hbm.at[:, pl.ds(0, CHB)], semo).wait()

    def compute(bufp, bufb2, obuf):
        @plsc.parallel_loop(0, VPC, unroll=16)
        def _(v):
            o = v * L
            b = bufb2[pl.ds(o, L)]
            r = plsc.load_gather(rbuf, [b])
            obuf[0, pl.ds(o, L)] = bufp[0, pl.ds(o, L)] * r
            obuf[1, pl.ds(o, L)] = bufp[1, pl.ds(o, L)] * r
            obuf[2, pl.ds(o, L)] = bufp[2, pl.ds(o, L)] * r

    start(0, bufa, bba, spa, sba)

    pltpu.make_async_copy(psum_hbm, psb, sps).wait()
    pltpu.make_async_copy(pcnt_hbm, pcb, spc).wait()

    def r_body(g, carry):
        sbase = g * L
        ssum = psb[0, pl.ds(sbase, L)]
        scnt = pcb[0, pl.ds(sbase, L)]
        for t in range(1, NW):
            ssum = ssum + psb[t, pl.ds(sbase, L)]
            scnt = scnt + pcb[t, pl.ds(sbase, L)]
        mean = ssum / jnp.maximum(scnt, onev)
        rbuf[pl.ds(sbase, L)] = w / (mean + epsv)
        return carry

    lax.fori_loop(0, S // L, r_body, 0)

    def pair_body(j, carry):
        k1 = 2 * j + 1
        k2 = 2 * j + 2

        @pl.when(k1 < n_my)
        def _():
            start(k1, bufb, bbb, spb, sbb)

        wait_in(bufa, bba, spa, sba)

        @pl.when(j > 0)
        def _():
            wait_out(oba, soa)

        compute(bufa, bba, oba)
        start_out(2 * j, oba, soa)

        @pl.when(k2 < n_my)
        def _():
            start(k2, bufa, bba, spa, sba)

        @pl.when(k1 < n_my)
        def _():
            wait_in(bufb, bbb, spb, sbb)

            @pl.when(j > 0)
            def _():
                wait_out(obb, sob)

            compute(bufb, bbb, obb)
            start_out(k1, obb, sob)

        return carry

    lax.fori_loop(0, (NFULL + 1) // 2, pair_body, 0)
    wait_out(oba, soa)
    wait_out(obb, sob)


def kernel(pos, batch, weight):
    pos_t = jnp.swapaxes(pos, 0, 1)
    wvec = jnp.broadcast_to(weight.reshape(1), (L,)).astype(jnp.float32)
    psum, pcnt = _pass1(pos_t, batch)
    out_t = _pass2(pos_t, batch, wvec, psum, pcnt)
    return jnp.swapaxes(out_t, 0, 1)


# speedup vs baseline: 1.3373x; 1.0110x over previous
import functools

import jax
import jax.numpy as jnp
from jax import lax
from jax.experimental import pallas as pl
from jax.experimental.pallas import tpu as pltpu
from jax.experimental.pallas import tpu_sc as plsc

N = 3200000
S = 1024
EPS = 1e-5
L = 16
NC, NS = 2, 16
NW = NC * NS
CHB = 3200
NCHT = N // CHB
VPC = CHB // L
NFULL = NCHT // NW
NEXTRA = NCHT % NW
CHB1 = 6400
NCHT1 = N // CHB1
VPC1 = CHB1 // L
NFULL1 = NCHT1 // NW
NEXTRA1 = NCHT1 % NW
SB = 256
SSTR = S + 1

_mesh = plsc.VectorSubcoreMesh(core_axis_name="c", subcore_axis_name="s")
_params = pltpu.CompilerParams(needs_layout_passes=False)


def _fast_norm(n2):
    i = lax.bitcast_convert_type(n2, jnp.int32)
    i = jnp.full((L,), 0x5F3759DF, jnp.int32) - lax.shift_right_logical(i, 1)
    y = lax.bitcast_convert_type(i, jnp.float32)
    ah = n2 * jnp.full((L,), 0.5, jnp.float32)
    c15 = jnp.full((L,), 1.5, jnp.float32)
    y = y * (c15 - ah * y * y)
    return n2 * y


@functools.partial(
    pl.kernel,
    mesh=_mesh,
    out_type=[
        jax.ShapeDtypeStruct((NW, S), jnp.float32),
        jax.ShapeDtypeStruct((NW, S), jnp.float32),
    ],
    scratch_types=[
        pltpu.VMEM((3, CHB1), jnp.float32),
        pltpu.VMEM((3, CHB1), jnp.float32),
        pltpu.VMEM((CHB1,), jnp.int32),
        pltpu.VMEM((CHB1,), jnp.int32),
        pltpu.VMEM((L * SSTR,), jnp.float32),
        pltpu.VMEM((L * SSTR,), jnp.float32),
        pltpu.VMEM((S,), jnp.float32),
        pltpu.VMEM((S,), jnp.float32),
        pltpu.SemaphoreType.DMA,
        pltpu.SemaphoreType.DMA,
        pltpu.SemaphoreType.DMA,
        pltpu.SemaphoreType.DMA,
    ],
    compiler_params=_params,
)
def _pass1(pos_hbm, batch_hbm, psum_hbm, pcnt_hbm,
           bufa, bufb, bba, bbb, accs, accc, reds, redc,
           spa, sba, spb, sbb):
    wid = lax.axis_index("s") * NC + lax.axis_index("c")
    lanes = lax.iota(jnp.int32, L)
    laneoff = lanes * SSTR
    zero = jnp.zeros((L,), jnp.float32)
    ones = jnp.ones((L,), jnp.float32)
    n_my = jnp.int32(NFULL1) + jnp.where(wid < NEXTRA1, 1, 0).astype(jnp.int32)

    @plsc.parallel_loop(0, SSTR, unroll=5)
    def _(i):
        accs[pl.ds(i * L, L)] = zero
        accc[pl.ds(i * L, L)] = zero

    def start(k, bufp, bufb2, semp, semb):
        base = (wid + k * NW) * CHB1
        pltpu.async_copy(pos_hbm.at[:, pl.ds(base, CHB1)], bufp, semp)
        pltpu.async_copy(batch_hbm.at[pl.ds(base, CHB1)], bufb2, semb)

    def wait(bufp, bufb2, semp, semb):
        pltpu.make_async_copy(pos_hbm.at[:, pl.ds(0, CHB1)], bufp, semp).wait()
        pltpu.make_async_copy(batch_hbm.at[pl.ds(0, CHB1)], bufb2, semb).wait()

    def compute(bufp, bufb2):
        @plsc.parallel_loop(0, VPC1, unroll=8)
        def _(v):
            o = v * L
            b = bufb2[pl.ds(o, L)]
            x = bufp[0, pl.ds(o, L)]
            y = bufp[1, pl.ds(o, L)]
            z = bufp[2, pl.ds(o, L)]
            nrm = _fast_norm(x * x + y * y + z * z)
            idx = b + laneoff
            plsc.addupdate_scatter(accs, [idx], nrm)
            plsc.addupdate_scatter(accc, [idx], ones)

    start(0, bufa, bba, spa, sba)

    def pair_body(j, carry):
        k1 = 2 * j + 1
        k2 = 2 * j + 2

        @pl.when(k1 < n_my)
        def _():
            start(k1, bufb, bbb, spb, sbb)

        wait(bufa, bba, spa, sba)
        compute(bufa, bba)

        @pl.when(k2 < n_my)
        def _():
            start(k2, bufa, bba, spa, sba)

        @pl.when(k1 < n_my)
        def _():
            wait(bufb, bbb, spb, sbb)
            compute(bufb, bbb)

        return carry

    lax.fori_loop(0, (NFULL1 + 1) // 2, pair_body, 0)

    def red_body(g, carry):
        sbase = g * L
        ssum = accs[pl.ds(sbase, L)]
        scnt = accc[pl.ds(sbase, L)]
        for c in range(1, L):
            ssum = ssum + accs[pl.ds(c * SSTR + sbase, L)]
            scnt = scnt + accc[pl.ds(c * SSTR + sbase, L)]
        reds[pl.ds(sbase, L)] = ssum
        redc[pl.ds(sbase, L)] = scnt
        return carry

    lax.fori_loop(0, S // L, red_body, 0)
    pltpu.sync_copy(reds, psum_hbm.at[wid])
    pltpu.sync_copy(redc, pcnt_hbm.at[wid])


@functools.partial(
    pl.kernel,
    mesh=_mesh,
    out_type=jax.ShapeDtypeStruct((3, N), jnp.float32),
    scratch_types=[
        pltpu.VMEM((NW, S), jnp.float32),
        pltpu.VMEM((NW, S), jnp.float32),
        pltpu.VMEM((S,), jnp.float32),
        pltpu.VMEM((L,), jnp.float32),
        pltpu.VMEM((3, CHB), jnp.float32),
        pltpu.VMEM((3, CHB), jnp.float32),
        pltpu.VMEM((CHB,), jnp.int32),
        pltpu.VMEM((CHB,), jnp.int32),
        pltpu.VMEM((3, CHB), jnp.float32),
        pltpu.VMEM((3, CHB), jnp.float32),
        pltpu.SemaphoreType.DMA,
        pltpu.SemaphoreType.DMA,
        pltpu.SemaphoreType.DMA,
        pltpu.SemaphoreType.DMA,
        pltpu.SemaphoreType.DMA,
        pltpu.SemaphoreType.DMA,
        pltpu.SemaphoreType.DMA,
        pltpu.SemaphoreType.DMA,
    ],
    compiler_params=_params,
)
def _pass2(pos_hbm, batch_hbm, w_hbm, psum_hbm, pcnt_hbm, out_hbm,
           psb, pcb, rbuf, wbuf, bufa, bufb, bba, bbb, oba, obb,
           spa, sba, spb, sbb, soa, sob, sps, spc):
    wid = lax.axis_index("s") * NC + lax.axis_index("c")
    onev = jnp.ones((L,), jnp.float32)
    epsv = jnp.full((L,), EPS, jnp.float32)
    n_my = jnp.int32(NFULL) + jnp.where(wid < NEXTRA, 1, 0).astype(jnp.int32)

    pltpu.async_copy(psum_hbm, psb, sps)
    pltpu.async_copy(pcnt_hbm, pcb, spc)
    pltpu.sync_copy(w_hbm, wbuf)
    w = wbuf[pl.ds(0, L)]

    def start(k, bufp, bufb2, semp, semb):
        base = (wid + k * NW) * CHB
        pltpu.async_copy(pos_hbm.at[:, pl.ds(base, CHB)], bufp, semp)
        pltpu.async_copy(batch_hbm.at[pl.ds(base, CHB)], bufb2, semb)

    def wait_in(bufp, bufb2, semp, semb):
        pltpu.make_async_copy(pos_hbm.at[:, pl.ds(0, CHB)], bufp, semp).wait()
        pltpu.make_async_copy(batch_hbm.at[pl.ds(0, CHB)], bufb2, semb).wait()

    def start_out(k, obuf, semo):
        base = (wid + k * NW) * CHB
        pltpu.async_copy(obuf, out_hbm.at[:, pl.ds(base, CHB)], semo)

    def wait_out(obuf, semo):
        pltpu.make_async_copy(obuf, out_hbm.at[:, pl.ds(0, CHB)], semo).wait()

    def compute(bufp, bufb2, obuf):
        @plsc.parallel_loop(0, VPC, unroll=8)
        def _(v):
            o = v * L
            b = bufb2[pl.ds(o, L)]
            r = plsc.load_gather(rbuf, [b])
            obuf[0, pl.ds(o, L)] = bufp[0, pl.ds(o, L)] * r
            obuf[1, pl.ds(o, L)] = bufp[1, pl.ds(o, L)] * r
            obuf[2, pl.ds(o, L)] = bufp[2, pl.ds(o, L)] * r

    start(0, bufa, bba, spa, sba)

    pltpu.make_async_copy(psum_hbm, psb, sps).wait()
    pltpu.make_async_copy(pcnt_hbm, pcb, spc).wait()

    def r_body(g, carry):
        sbase = g * L
        ssum = psb[0, pl.ds(sbase, L)]
        scnt = pcb[0, pl.ds(sbase, L)]
        for t in range(1, NW):
            ssum = ssum + psb[t, pl.ds(sbase, L)]
            scnt = scnt + pcb[t, pl.ds(sbase, L)]
        mean = ssum / jnp.maximum(scnt, onev)
        rbuf[pl.ds(sbase, L)] = w / (mean + epsv)
        return carry

    lax.fori_loop(0, S // L, r_body, 0)

    def pair_body(j, carry):
        k1 = 2 * j + 1
        k2 = 2 * j + 2

        @pl.when(k1 < n_my)
        def _():
            start(k1, bufb, bbb, spb, sbb)

        wait_in(bufa, bba, spa, sba)

        @pl.when(j > 0)
        def _():
            wait_out(oba, soa)

        compute(bufa, bba, oba)
        start_out(2 * j, oba, soa)

        @pl.when(k2 < n_my)
        def _():
            start(k2, bufa, bba, spa, sba)

        @pl.when(k1 < n_my)
        def _():
            wait_in(bufb, bbb, spb, sbb)

            @pl.when(j > 0)
            def _():
                wait_out(obb, sob)

            compute(bufb, bbb, obb)
            start_out(k1, obb, sob)

        return carry

    lax.fori_loop(0, (NFULL + 1) // 2, pair_body, 0)
    wait_out(oba, soa)
    wait_out(obb, sob)


def kernel(pos, batch, weight):
    pos_t = jnp.swapaxes(pos, 0, 1)
    wvec = jnp.broadcast_to(weight.reshape(1), (L,)).astype(jnp.float32)
    psum, pcnt = _pass1(pos_t, batch)
    out_t = _pass2(pos_t, batch, wvec, psum, pcnt)
    return jnp.swapaxes(out_t, 0, 1)


# speedup vs baseline: 1.3397x; 1.0017x over previous
import functools

import jax
import jax.numpy as jnp
from jax import lax
from jax.experimental import pallas as pl
from jax.experimental.pallas import tpu as pltpu
from jax.experimental.pallas import tpu_sc as plsc

N = 3200000
S = 1024
EPS = 1e-5
L = 16
NC, NS = 2, 16
NW = NC * NS
CHB = 3200
NCHT = N // CHB
VPC = CHB // L
NFULL = NCHT // NW
NEXTRA = NCHT % NW
CHB1 = 6400
NCHT1 = N // CHB1
VPC1 = CHB1 // L
NFULL1 = NCHT1 // NW
NEXTRA1 = NCHT1 % NW
SB = 256
SSTR = S + 1

_mesh = plsc.VectorSubcoreMesh(core_axis_name="c", subcore_axis_name="s")
_params = pltpu.CompilerParams(needs_layout_passes=False)


def _fast_norm(n2):
    i = lax.bitcast_convert_type(n2, jnp.int32)
    i = jnp.full((L,), 0x5F3759DF, jnp.int32) - lax.shift_right_logical(i, 1)
    y = lax.bitcast_convert_type(i, jnp.float32)
    ah = n2 * jnp.full((L,), 0.5, jnp.float32)
    c15 = jnp.full((L,), 1.5, jnp.float32)
    y = y * (c15 - ah * y * y)
    y = y * (c15 - ah * y * y)
    return n2 * y


@functools.partial(
    pl.kernel,
    mesh=_mesh,
    out_type=[
        jax.ShapeDtypeStruct((NW, S), jnp.float32),
        jax.ShapeDtypeStruct((NW, S), jnp.float32),
    ],
    scratch_types=[
        pltpu.VMEM((3, CHB1), jnp.float32),
        pltpu.VMEM((3, CHB1), jnp.float32),
        pltpu.VMEM((CHB1,), jnp.int32),
        pltpu.VMEM((CHB1,), jnp.int32),
        pltpu.VMEM((L * SSTR,), jnp.float32),
        pltpu.VMEM((L * SSTR,), jnp.float32),
        pltpu.VMEM((S,), jnp.float32),
        pltpu.VMEM((S,), jnp.float32),
        pltpu.SemaphoreType.DMA,
        pltpu.SemaphoreType.DMA,
        pltpu.SemaphoreType.DMA,
        pltpu.SemaphoreType.DMA,
    ],
    compiler_params=_params,
)
def _pass1(pos_hbm, batch_hbm, psum_hbm, pcnt_hbm,
           bufa, bufb, bba, bbb, accs, accc, reds, redc,
           spa, sba, spb, sbb):
    wid = lax.axis_index("s") * NC + lax.axis_index("c")
    lanes = lax.iota(jnp.int32, L)
    laneoff = lanes * SSTR
    zero = jnp.zeros((L,), jnp.float32)
    ones = jnp.ones((L,), jnp.float32)
    n_my = jnp.int32(NFULL1) + jnp.where(wid < NEXTRA1, 1, 0).astype(jnp.int32)

    @plsc.parallel_loop(0, SSTR, unroll=5)
    def _(i):
        accs[pl.ds(i * L, L)] = zero
        accc[pl.ds(i * L, L)] = zero

    def start(k, bufp, bufb2, semp, semb):
        base = (wid + k * NW) * CHB1
        pltpu.async_copy(pos_hbm.at[:, pl.ds(base, CHB1)], bufp, semp)
        pltpu.async_copy(batch_hbm.at[pl.ds(base, CHB1)], bufb2, semb)

    def wait(bufp, bufb2, semp, semb):
        pltpu.make_async_copy(pos_hbm.at[:, pl.ds(0, CHB1)], bufp, semp).wait()
        pltpu.make_async_copy(batch_hbm.at[pl.ds(0, CHB1)], bufb2, semb).wait()

    def compute(bufp, bufb2):
        @plsc.parallel_loop(0, VPC1, unroll=8)
        def _(v):
            o = v * L
            b = bufb2[pl.ds(o, L)]
            x = bufp[0, pl.ds(o, L)]
            y = bufp[1, pl.ds(o, L)]
            z = bufp[2, pl.ds(o, L)]
            nrm = _fast_norm(x * x + y * y + z * z)
            idx = b + laneoff
            plsc.addupdate_scatter(accs, [idx], nrm)
            plsc.addupdate_scatter(accc, [idx], ones)

    start(0, bufa, bba, spa, sba)

    def pair_body(j, carry):
        k1 = 2 * j + 1
        k2 = 2 * j + 2

        @pl.when(k1 < n_my)
        def _():
            start(k1, bufb, bbb, spb, sbb)

        wait(bufa, bba, spa, sba)
        compute(bufa, bba)

        @pl.when(k2 < n_my)
        def _():
            start(k2, bufa, bba, spa, sba)

        @pl.when(k1 < n_my)
        def _():
            wait(bufb, bbb, spb, sbb)
            compute(bufb, bbb)

        return carry

    lax.fori_loop(0, (NFULL1 + 1) // 2, pair_body, 0)

    def red_body(g, carry):
        sbase = g * L
        ssum = accs[pl.ds(sbase, L)]
        scnt = accc[pl.ds(sbase, L)]
        for c in range(1, L):
            ssum = ssum + accs[pl.ds(c * SSTR + sbase, L)]
            scnt = scnt + accc[pl.ds(c * SSTR + sbase, L)]
        reds[pl.ds(sbase, L)] = ssum
        redc[pl.ds(sbase, L)] = scnt
        return carry

    lax.fori_loop(0, S // L, red_body, 0)
    pltpu.sync_copy(reds, psum_hbm.at[wid])
    pltpu.sync_copy(redc, pcnt_hbm.at[wid])


@functools.partial(
    pl.kernel,
    mesh=_mesh,
    out_type=jax.ShapeDtypeStruct((3, N), jnp.float32),
    scratch_types=[
        pltpu.VMEM((NW, S), jnp.float32),
        pltpu.VMEM((NW, S), jnp.float32),
        pltpu.VMEM((S,), jnp.float32),
        pltpu.VMEM((L,), jnp.float32),
        pltpu.VMEM((3, CHB), jnp.float32),
        pltpu.VMEM((3, CHB), jnp.float32),
        pltpu.VMEM((CHB,), jnp.int32),
        pltpu.VMEM((CHB,), jnp.int32),
        pltpu.VMEM((3, CHB), jnp.float32),
        pltpu.VMEM((3, CHB), jnp.float32),
        pltpu.SemaphoreType.DMA,
        pltpu.SemaphoreType.DMA,
        pltpu.SemaphoreType.DMA,
        pltpu.SemaphoreType.DMA,
        pltpu.SemaphoreType.DMA,
        pltpu.SemaphoreType.DMA,
        pltpu.SemaphoreType.DMA,
        pltpu.SemaphoreType.DMA,
    ],
    compiler_params=_params,
)
def _pass2(pos_hbm, batch_hbm, w_hbm, psum_hbm, pcnt_hbm, out_hbm,
           psb, pcb, rbuf, wbuf, bufa, bufb, bba, bbb, oba, obb,
           spa, sba, spb, sbb, soa, sob, sps, spc):
    wid = lax.axis_index("s") * NC + lax.axis_index("c")
    onev = jnp.ones((L,), jnp.float32)
    epsv = jnp.full((L,), EPS, jnp.float32)
    n_my = jnp.int32(NFULL) + jnp.where(wid < NEXTRA, 1, 0).astype(jnp.int32)

    pltpu.async_copy(psum_hbm, psb, sps)
    pltpu.async_copy(pcnt_hbm, pcb, spc)
    pltpu.sync_copy(w_hbm, wbuf)
    w = wbuf[pl.ds(0, L)]

    def start(k, bufp, bufb2, semp, semb):
        base = (wid + k * NW) * CHB
        pltpu.async_copy(pos_hbm.at[:, pl.ds(base, CHB)], bufp, semp)
        pltpu.async_copy(batch_hbm.at[pl.ds(base, CHB)], bufb2, semb)

    def wait_in(bufp, bufb2, semp, semb):
        pltpu.make_async_copy(pos_hbm.at[:, pl.ds(0, CHB)], bufp, semp).wait()
        pltpu.make_async_copy(batch_hbm.at[pl.ds(0, CHB)], bufb2, semb).wait()

    def start_out(k, obuf, semo):
        base = (wid + k * NW) * CHB
        pltpu.async_copy(obuf, out_hbm.at[:, pl.ds(base, CHB)], semo)

    def wait_out(obuf, semo):
        pltpu.make_async_copy(obuf, out_hbm.at[:, pl.ds(0, CHB)], semo).wait()

    def compute(bufp, bufb2, obuf):
        @plsc.parallel_loop(0, VPC, unroll=8)
        def _(v):
            o = v * L
            b = bufb2[pl.ds(o, L)]
            r = plsc.load_gather(rbuf, [b])
            obuf[0, pl.ds(o, L)] = bufp[0, pl.ds(o, L)] * r
            obuf[1, pl.ds(o, L)] = bufp[1, pl.ds(o, L)] * r
            obuf[2, pl.ds(o, L)] = bufp[2, pl.ds(o, L)] * r

    start(0, bufa, bba, spa, sba)

    pltpu.make_async_copy(psum_hbm, psb, sps).wait()
    pltpu.make_async_copy(pcnt_hbm, pcb, spc).wait()

    def r_body(g, carry):
        sbase = g * L
        ssum = psb[0, pl.ds(sbase, L)]
        scnt = pcb[0, pl.ds(sbase, L)]
        for t in range(1, NW):
            ssum = ssum + psb[t, pl.ds(sbase, L)]
            scnt = scnt + pcb[t, pl.ds(sbase, L)]
        mean = ssum / jnp.maximum(scnt, onev)
        rbuf[pl.ds(sbase, L)] = w / (mean + epsv)
        return carry

    lax.fori_loop(0, S // L, r_body, 0)

    def pair_body(j, carry):
        k1 = 2 * j + 1
        k2 = 2 * j + 2

        @pl.when(k1 < n_my)
        def _():
            start(k1, bufb, bbb, spb, sbb)

        wait_in(bufa, bba, spa, sba)

        @pl.when(j > 0)
        def _():
            wait_out(oba, soa)

        compute(bufa, bba, oba)
        start_out(2 * j, oba, soa)

        @pl.when(k2 < n_my)
        def _():
            start(k2, bufa, bba, spa, sba)

        @pl.when(k1 < n_my)
        def _():
            wait_in(bufb, bbb, spb, sbb)

            @pl.when(j > 0)
            def _():
                wait_out(obb, sob)

            compute(bufb, bbb, obb)
            start_out(k1, obb, sob)

        return carry

    lax.fori_loop(0, (NFULL + 1) // 2, pair_body, 0)
    wait_out(oba, soa)
    wait_out(obb, sob)


def kernel(pos, batch, weight):
    pos_t = jnp.swapaxes(pos, 0, 1)
    wvec = jnp.broadcast_to(weight.reshape(1), (L,)).astype(jnp.float32)
    psum, pcnt = _pass1(pos_t, batch)
    out_t = _pass2(pos_t, batch, wvec, psum, pcnt)
    return jnp.swapaxes(out_t, 0, 1)


# speedup vs baseline: 1.3418x; 1.0016x over previous
import functools

import jax
import jax.numpy as jnp
from jax import lax
from jax.experimental import pallas as pl
from jax.experimental.pallas import tpu as pltpu
from jax.experimental.pallas import tpu_sc as plsc

N = 3200000
S = 1024
EPS = 1e-5
L = 16
NC, NS = 2, 16
NW = NC * NS
CHB = 3200
NCHT = N // CHB
VPC = CHB // L
NFULL = NCHT // NW
NEXTRA = NCHT % NW
CHB1 = 6400
NCHT1 = N // CHB1
VPC1 = CHB1 // L
NFULL1 = NCHT1 // NW
NEXTRA1 = NCHT1 % NW
SSTR = S + 1

_mesh = plsc.VectorSubcoreMesh(core_axis_name="c", subcore_axis_name="s")
_params = pltpu.CompilerParams(needs_layout_passes=False)


def _fast_norm(n2):
    i = lax.bitcast_convert_type(n2, jnp.int32)
    i = jnp.full((L,), 0x5F3759DF, jnp.int32) - lax.shift_right_logical(i, 1)
    y = lax.bitcast_convert_type(i, jnp.float32)
    ah = n2 * jnp.full((L,), 0.5, jnp.float32)
    c15 = jnp.full((L,), 1.5, jnp.float32)
    y = y * (c15 - ah * y * y)
    y = y * (c15 - ah * y * y)
    return n2 * y


@functools.partial(
    pl.kernel,
    mesh=_mesh,
    out_type=[
        jax.ShapeDtypeStruct((NW, S), jnp.float32),
        jax.ShapeDtypeStruct((NW, S), jnp.float32),
    ],
    scratch_types=[
        pltpu.VMEM((3, CHB1), jnp.float32),
        pltpu.VMEM((3, CHB1), jnp.float32),
        pltpu.VMEM((CHB1,), jnp.int32),
        pltpu.VMEM((CHB1,), jnp.int32),
        pltpu.VMEM((L * SSTR,), jnp.float32),
        pltpu.VMEM((L * SSTR,), jnp.float32),
        pltpu.VMEM((S,), jnp.float32),
        pltpu.VMEM((S,), jnp.float32),
        pltpu.SemaphoreType.DMA,
        pltpu.SemaphoreType.DMA,
        pltpu.SemaphoreType.DMA,
        pltpu.SemaphoreType.DMA,
    ],
    compiler_params=_params,
)
def _pass1(pos_hbm, batch_hbm, psum_hbm, pcnt_hbm,
           bufa, bufb, bba, bbb, accs, accc, reds, redc,
           spa, sba, spb, sbb):
    wid = lax.axis_index("s") * NC + lax.axis_index("c")
    lanes = lax.iota(jnp.int32, L)
    laneoff = lanes * SSTR
    zero = jnp.zeros((L,), jnp.float32)
    ones = jnp.ones((L,), jnp.float32)
    n_my = jnp.int32(NFULL1) + jnp.where(wid < NEXTRA1, 1, 0).astype(jnp.int32)

    @plsc.parallel_loop(0, SSTR, unroll=5)
    def _(i):
        accs[pl.ds(i * L, L)] = zero
        accc[pl.ds(i * L, L)] = zero

    def start(k, bufp, bufb2, semp, semb):
        base = (wid + k * NW) * CHB1
        pltpu.async_copy(pos_hbm.at[:, pl.ds(base, CHB1)], bufp, semp)
        pltpu.async_copy(batch_hbm.at[pl.ds(base, CHB1)], bufb2, semb)

    def wait(bufp, bufb2, semp, semb):
        pltpu.make_async_copy(pos_hbm.at[:, pl.ds(0, CHB1)], bufp, semp).wait()
        pltpu.make_async_copy(batch_hbm.at[pl.ds(0, CHB1)], bufb2, semb).wait()

    def compute(bufp, bufb2):
        @plsc.parallel_loop(0, VPC1, unroll=8)
        def _(v):
            o = v * L
            b = bufb2[pl.ds(o, L)]
            x = bufp[0, pl.ds(o, L)]
            y = bufp[1, pl.ds(o, L)]
            z = bufp[2, pl.ds(o, L)]
            nrm = _fast_norm(x * x + y * y + z * z)
            idx = b + laneoff
            plsc.addupdate_scatter(accs, [idx], nrm)
            plsc.addupdate_scatter(accc, [idx], ones)

    start(0, bufa, bba, spa, sba)

    def pair_body(j, carry):
        k1 = 2 * j + 1
        k2 = 2 * j + 2

        @pl.when(k1 < n_my)
        def _():
            start(k1, bufb, bbb, spb, sbb)

        wait(bufa, bba, spa, sba)
        compute(bufa, bba)

        @pl.when(k2 < n_my)
        def _():
            start(k2, bufa, bba, spa, sba)

        @pl.when(k1 < n_my)
        def _():
            wait(bufb, bbb, spb, sbb)
            compute(bufb, bbb)

        return carry

    lax.fori_loop(0, (NFULL1 + 1) // 2, pair_body, 0)

    def red_body(g, carry):
        sbase = g * L
        ssum = accs[pl.ds(sbase, L)]
        scnt = accc[pl.ds(sbase, L)]
        for c in range(1, L):
            ssum = ssum + accs[pl.ds(c * SSTR + sbase, L)]
            scnt = scnt + accc[pl.ds(c * SSTR + sbase, L)]
        reds[pl.ds(sbase, L)] = ssum
        redc[pl.ds(sbase, L)] = scnt
        return carry

    lax.fori_loop(0, S // L, red_body, 0)
    pltpu.sync_copy(reds, psum_hbm.at[wid])
    pltpu.sync_copy(redc, pcnt_hbm.at[wid])


@functools.partial(
    pl.kernel,
    mesh=_mesh,
    out_type=jax.ShapeDtypeStruct((3, N), jnp.float32),
    scratch_types=[
        pltpu.VMEM((NW, S), jnp.float32),
        pltpu.VMEM((NW, S), jnp.float32),
        pltpu.VMEM((S,), jnp.float32),
        pltpu.VMEM((L,), jnp.float32),
        pltpu.VMEM((3, CHB), jnp.float32),
        pltpu.VMEM((3, CHB), jnp.float32),
        pltpu.VMEM((CHB,), jnp.int32),
        pltpu.VMEM((CHB,), jnp.int32),
        pltpu.VMEM((3, CHB), jnp.float32),
        pltpu.VMEM((3, CHB), jnp.float32),
        pltpu.SemaphoreType.DMA,
        pltpu.SemaphoreType.DMA,
        pltpu.SemaphoreType.DMA,
        pltpu.SemaphoreType.DMA,
        pltpu.SemaphoreType.DMA,
        pltpu.SemaphoreType.DMA,
        pltpu.SemaphoreType.DMA,
        pltpu.SemaphoreType.DMA,
    ],
    compiler_params=_params,
)
def _pass2(pos_hbm, batch_hbm, w_hbm, psum_hbm, pcnt_hbm, out_hbm,
           psb, pcb, rbuf, wbuf, bufa, bufb, bba, bbb, oba, obb,
           spa, sba, spb, sbb, soa, sob, sps, spc):
    wid = lax.axis_index("s") * NC + lax.axis_index("c")
    onev = jnp.ones((L,), jnp.float32)
    epsv = jnp.full((L,), EPS, jnp.float32)
    n_my = jnp.int32(NFULL) + jnp.where(wid < NEXTRA, 1, 0).astype(jnp.int32)

    pltpu.async_copy(psum_hbm, psb, sps)
    pltpu.async_copy(pcnt_hbm, pcb, spc)
    pltpu.sync_copy(w_hbm, wbuf)
    w = wbuf[pl.ds(0, L)]

    def start(k, bufp, bufb2, semp, semb):
        base = (wid + k * NW) * CHB
        pltpu.async_copy(pos_hbm.at[:, pl.ds(base, CHB)], bufp, semp)
        pltpu.async_copy(batch_hbm.at[pl.ds(base, CHB)], bufb2, semb)

    def wait_in(bufp, bufb2, semp, semb):
        pltpu.make_async_copy(pos_hbm.at[:, pl.ds(0, CHB)], bufp, semp).wait()
        pltpu.make_async_copy(batch_hbm.at[pl.ds(0, CHB)], bufb2, semb).wait()

    def start_out(k, obuf, semo):
        base = (wid + k * NW) * CHB
        pltpu.async_copy(obuf, out_hbm.at[:, pl.ds(base, CHB)], semo)

    def wait_out(obuf, semo):
        pltpu.make_async_copy(obuf, out_hbm.at[:, pl.ds(0, CHB)], semo).wait()

    def compute(bufp, bufb2, obuf):
        @plsc.parallel_loop(0, VPC, unroll=8)
        def _(v):
            o = v * L
            b = bufb2[pl.ds(o, L)]
            r = plsc.load_gather(rbuf, [b])
            obuf[0, pl.ds(o, L)] = bufp[0, pl.ds(o, L)] * r
            obuf[1, pl.ds(o, L)] = bufp[1, pl.ds(o, L)] * r
            obuf[2, pl.ds(o, L)] = bufp[2, pl.ds(o, L)] * r

    start(0, bufa, bba, spa, sba)

    pltpu.make_async_copy(psum_hbm, psb, sps).wait()
    pltpu.make_async_copy(pcnt_hbm, pcb, spc).wait()

    def r_body(g, carry):
        sbase = g * L
        ssum = psb[0, pl.ds(sbase, L)]
        scnt = pcb[0, pl.ds(sbase, L)]
        for t in range(1, NW):
            ssum = ssum + psb[t, pl.ds(sbase, L)]
            scnt = scnt + pcb[t, pl.ds(sbase, L)]
        mean = ssum / jnp.maximum(scnt, onev)
        rbuf[pl.ds(sbase, L)] = w / (mean + epsv)
        return carry

    lax.fori_loop(0, S // L, r_body, 0)

    def pair_body(j, carry):
        k1 = 2 * j + 1
        k2 = 2 * j + 2

        @pl.when(k1 < n_my)
        def _():
            start(k1, bufb, bbb, spb, sbb)

        wait_in(bufa, bba, spa, sba)

        @pl.when(j > 0)
        def _():
            wait_out(oba, soa)

        compute(bufa, bba, oba)
        start_out(2 * j, oba, soa)

        @pl.when(k2 < n_my)
        def _():
            start(k2, bufa, bba, spa, sba)

        @pl.when(k1 < n_my)
        def _():
            wait_in(bufb, bbb, spb, sbb)

            @pl.when(j > 0)
            def _():
                wait_out(obb, sob)

            compute(bufb, bbb, obb)
            start_out(k1, obb, sob)

        return carry

    lax.fori_loop(0, (NFULL + 1) // 2, pair_body, 0)
    wait_out(oba, soa)
    wait_out(obb, sob)


def kernel(pos, batch, weight):
    pos_t = jnp.swapaxes(pos, 0, 1)
    wvec = jnp.broadcast_to(weight.reshape(1), (L,)).astype(jnp.float32)
    psum, pcnt = _pass1(pos_t, batch)
    out_t = _pass2(pos_t, batch, wvec, psum, pcnt)
    return jnp.swapaxes(out_t, 0, 1)
